# baseline (speedup 1.0000x reference)
import functools

import jax
import jax.numpy as jnp
from jax import lax
from jax.experimental import pallas as pl
from jax.experimental.pallas import tpu as pltpu
from jax.experimental.pallas import tpu_sc as plsc

N = 10000
E = 160000
D = 256
DH = 128
NSC = 16
N_PAD = 10240
R = N_PAD // NSC
EPT = E // NSC
B = 80
NCHUNK = EPT // B

BM = 400


def _mm_y_body(x_ref, wn_ref, ya_ref, yb_ref):
    y = jnp.dot(x_ref[...], wn_ref[...], preferred_element_type=jnp.float32)
    ya_ref[...] = y[:, :DH]
    yb_ref[...] = y[:, DH:]


def _mm_y(x, wn):
    return pl.pallas_call(
        _mm_y_body,
        grid=(N // BM,),
        in_specs=[
            pl.BlockSpec((BM, D), lambda i: (i, 0)),
            pl.BlockSpec((D, D), lambda i: (0, 0)),
        ],
        out_specs=[
            pl.BlockSpec((BM, DH), lambda i: (i, 0)),
            pl.BlockSpec((BM, DH), lambda i: (i, 0)),
        ],
        out_shape=[
            jax.ShapeDtypeStruct((N_PAD, DH), jnp.float32),
            jax.ShapeDtypeStruct((N_PAD, DH), jnp.float32),
        ],
    )(x, wn)


def _mm_s_body(x_ref, ws_ref, b_ref, s_ref):
    s_ref[...] = jnp.dot(x_ref[...], ws_ref[...], preferred_element_type=jnp.float32) + b_ref[...]


def _mm_s(x, ws, b):
    return pl.pallas_call(
        _mm_s_body,
        grid=(N // BM,),
        in_specs=[
            pl.BlockSpec((BM, D), lambda i: (i, 0)),
            pl.BlockSpec((D, D), lambda i: (0, 0)),
            pl.BlockSpec((1, D), lambda i: (0, 0)),
        ],
        out_specs=pl.BlockSpec((BM, D), lambda i: (i, 0)),
        out_shape=jax.ShapeDtypeStruct((N_PAD, D), jnp.float32),
    )(x, ws, b.reshape(1, D))


def _h_of(s1_ref, aa_ref, ab_ref, c0_ref, c1_ref):
    inv = 1.0 / jnp.maximum(c0_ref[...] + c1_ref[...], 1.0)
    agg = jnp.concatenate([aa_ref[...], ab_ref[...]], axis=1) * inv
    return jnp.maximum(s1_ref[...] + agg, 0.0)


def _mid_y_body(s1_ref, aa_ref, ab_ref, c0_ref, c1_ref, wn_ref, ya_ref, yb_ref):
    h = _h_of(s1_ref, aa_ref, ab_ref, c0_ref, c1_ref)
    y = jnp.dot(h, wn_ref[...], preferred_element_type=jnp.float32)
    ya_ref[...] = y[:, :DH]
    yb_ref[...] = y[:, DH:]


def _mid_s_body(s1_ref, aa_ref, ab_ref, c0_ref, c1_ref, ws_ref, b_ref, s2_ref):
    h = _h_of(s1_ref, aa_ref, ab_ref, c0_ref, c1_ref)
    s2_ref[...] = jnp.dot(h, ws_ref[...], preferred_element_type=jnp.float32) + b_ref[...]


_MID_IN = [
    pl.BlockSpec((BM, D), lambda i: (i, 0)),
    pl.BlockSpec((BM, DH), lambda i: (i, 0)),
    pl.BlockSpec((BM, DH), lambda i: (i, 0)),
    pl.BlockSpec((BM, 1), lambda i: (i, 0)),
    pl.BlockSpec((BM, 1), lambda i: (i, 0)),
    pl.BlockSpec((D, D), lambda i: (0, 0)),
]


def _mid_y(s1, aa, ab, c0, c1, wn):
    return pl.pallas_call(
        _mid_y_body,
        grid=(N // BM,),
        in_specs=_MID_IN,
        out_specs=[
            pl.BlockSpec((BM, DH), lambda i: (i, 0)),
            pl.BlockSpec((BM, DH), lambda i: (i, 0)),
        ],
        out_shape=[
            jax.ShapeDtypeStruct((N_PAD, DH), jnp.float32),
            jax.ShapeDtypeStruct((N_PAD, DH), jnp.float32),
        ],
    )(s1, aa, ab, c0, c1, wn)


def _mid_s(s1, aa, ab, c0, c1, ws, b):
    return pl.pallas_call(
        _mid_s_body,
        grid=(N // BM,),
        in_specs=_MID_IN + [pl.BlockSpec((1, D), lambda i: (0, 0))],
        out_specs=pl.BlockSpec((BM, D), lambda i: (i, 0)),
        out_shape=jax.ShapeDtypeStruct((N_PAD, D), jnp.float32),
    )(s1, aa, ab, c0, c1, ws, b.reshape(1, D))


def _fin_body(s2_ref, aa_ref, ab_ref, c0_ref, c1_ref, o_ref):
    inv = 1.0 / jnp.maximum(c0_ref[...] + c1_ref[...], 1.0)
    o_ref[...] = s2_ref[...] + jnp.concatenate([aa_ref[...], ab_ref[...]], axis=1) * inv


def _fin(s2, aa, ab, c0, c1):
    return pl.pallas_call(
        _fin_body,
        grid=(N // BM,),
        in_specs=[
            pl.BlockSpec((BM, D), lambda i: (i, 0)),
            pl.BlockSpec((BM, DH), lambda i: (i, 0)),
            pl.BlockSpec((BM, DH), lambda i: (i, 0)),
            pl.BlockSpec((BM, 1), lambda i: (i, 0)),
            pl.BlockSpec((BM, 1), lambda i: (i, 0)),
        ],
        out_specs=pl.BlockSpec((BM, D), lambda i: (i, 0)),
        out_shape=jax.ShapeDtypeStruct((N, D), jnp.float32),
    )(s2, aa, ab, c0, c1)


ECNT = E // 32


@functools.lru_cache(maxsize=None)
def _make_segsum():
    mesh = plsc.VectorSubcoreMesh(
        core_axis_name="c", subcore_axis_name="s", num_cores=2, num_subcores=NSC)

    out_type = [
        jax.ShapeDtypeStruct((N_PAD, DH), jnp.float32),
        jax.ShapeDtypeStruct((N_PAD, DH), jnp.float32),
    ]
    scratch = [
        pltpu.VMEM((EPT,), jnp.int32),
        pltpu.VMEM((2, B), jnp.int32),
        pltpu.VMEM((2, B, DH), jnp.float32),
        pltpu.VMEM_SHARED((N_PAD, DH), jnp.float32),
        pltpu.SemaphoreType.DMA,
        pltpu.SemaphoreType.DMA,
        pltpu.SemaphoreType.DMA,
    ]

    def body(ya, yb, src2, dst1, zrow, agg_a, agg_b,
             sidx, didx, rows, accum, gsem, ssem, dsem):
        c = lax.axis_index("c")
        s = lax.axis_index("s")
        row0 = s * R
        ebase = s * EPT

        pltpu.sync_copy(zrow.at[pl.ds(row0, R)], accum.at[pl.ds(row0, R)])
        pltpu.sync_copy(src2.at[s], sidx)
        plsc.subcore_barrier()

        def gather(i, buf):
            @pl.when(c == 0)
            def _():
                pltpu.async_copy(ya.at[sidx.at[pl.ds(i * B, B)]], rows.at[buf], gsem)

            @pl.when(c == 1)
            def _():
                pltpu.async_copy(yb.at[sidx.at[pl.ds(i * B, B)]], rows.at[buf], gsem)

        pltpu.async_copy(dst1.at[pl.ds(ebase, B)], didx.at[0], dsem)
        gather(0, 0)

        def step(i, carry):
            cur = lax.rem(i, 2)
            nxt = 1 - cur

            @pl.when(i > 0)
            def _():
                pltpu.make_async_copy(rows.at[nxt], accum.at[didx.at[nxt]], ssem).wait()

            @pl.when(i + 1 < NCHUNK)
            def _():
                pltpu.async_copy(dst1.at[pl.ds(ebase + (i + 1) * B, B)], didx.at[nxt], dsem)
                gather(i + 1, nxt)

            pltpu.make_async_copy(dst1.at[pl.ds(ebase, B)], didx.at[cur], dsem).wait()
            pltpu.make_async_copy(ya.at[sidx.at[pl.ds(0, B)]], rows.at[cur], gsem).wait()
            pltpu.async_copy(rows.at[cur], accum.at[didx.at[cur]], ssem, add=True)
            return carry

        lax.fori_loop(0, NCHUNK, step, 0)
        pltpu.make_async_copy(rows.at[0], accum.at[didx.at[0]], ssem).wait()
        plsc.subcore_barrier()

        @pl.when(c == 0)
        def _():
            pltpu.sync_copy(accum.at[pl.ds(row0, R)], agg_a.at[pl.ds(row0, R)])

        @pl.when(c == 1)
        def _():
            pltpu.sync_copy(accum.at[pl.ds(row0, R)], agg_b.at[pl.ds(row0, R)])

    return pl.kernel(
        body, out_type=out_type, mesh=mesh, scratch_types=scratch,
        compiler_params=pltpu.CompilerParams(needs_layout_passes=False))


@functools.lru_cache(maxsize=None)
def _make_cnt():
    mesh = plsc.VectorSubcoreMesh(
        core_axis_name="c", subcore_axis_name="s", num_cores=2, num_subcores=NSC)

    out_type = [
        jax.ShapeDtypeStruct((N_PAD,), jnp.float32),
        jax.ShapeDtypeStruct((N_PAD,), jnp.float32),
    ]
    scratch = [
        pltpu.VMEM((ECNT,), jnp.int32),
        pltpu.VMEM((N_PAD,), jnp.float32),
        pltpu.VMEM((NSC, R), jnp.float32),
        pltpu.VMEM((R,), jnp.float32),
        pltpu.VMEM_SHARED((NSC, N_PAD), jnp.float32),
    ]

    def body(dst2, c0_out, c1_out, didx, cntp, cred, cout, cnt_all):
        c = lax.axis_index("c")
        s = lax.axis_index("s")
        w = c * NSC + s
        row0 = s * R

        pltpu.sync_copy(dst2.at[w], didx)

        def _zc(i, carry):
            cntp[pl.ds(i * 16, 16)] = jnp.zeros((16,), jnp.float32)
            return carry
        lax.fori_loop(0, N_PAD // 16, _zc, 0)

        ones16 = jnp.ones((16,), jnp.float32)

        def _cc(i, carry):
            d16 = didx[pl.ds(i * 16, 16)]
            plsc.addupdate_scatter(cntp, [d16], ones16)
            return carry
        lax.fori_loop(0, ECNT // 16, _cc, 0)
        d16 = didx[pl.ds(ECNT - 16, 16)]
        lanes = lax.broadcasted_iota(jnp.int32, (16,), 0)
        plsc.addupdate_scatter(cntp, [d16], ones16, mask=lanes >= 8)

        pltpu.sync_copy(cntp, cnt_all.at[s])
        plsc.subcore_barrier()
        pltpu.sync_copy(cnt_all.at[:, pl.ds(row0, R)], cred)

        def red(j, carry):
            acc = jnp.zeros((16,), jnp.float32)
            for r in range(NSC):
                acc = acc + cred[r, pl.ds(j * 16, 16)]
            cout[pl.ds(j * 16, 16)] = acc
            return carry
        lax.fori_loop(0, R // 16, red, 0)

        @pl.when(c == 0)
        def _():
            pltpu.sync_copy(cout, c0_out.at[pl.ds(row0, R)])

        @pl.when(c == 1)
        def _():
            pltpu.sync_copy(cout, c1_out.at[pl.ds(row0, R)])

    return pl.kernel(
        body, out_type=out_type, mesh=mesh, scratch_types=scratch,
        compiler_params=pltpu.CompilerParams(needs_layout_passes=False))


@jax.jit
def kernel(x, edge_index, W1_self, W1_neigh, b1, W2_self, W2_neigh, b2):
    src2 = edge_index[0].reshape(NSC, EPT)
    dst1 = edge_index[1]
    dst2 = edge_index[1].reshape(2 * NSC, ECNT)
    zrow = jnp.zeros((N_PAD, DH), jnp.float32)

    cnt0, cnt1 = _make_cnt()(dst2)
    c0 = cnt0.reshape(N_PAD, 1)
    c1 = cnt1.reshape(N_PAD, 1)
    y1a, y1b = _mm_y(x, W1_neigh)
    agg_a, agg_b = _make_segsum()(y1a, y1b, src2, dst1, zrow)
    s1 = _mm_s(x, W1_self, b1)
    y2a, y2b = _mid_y(s1, agg_a, agg_b, c0, c1, W2_neigh)
    agg_a2, agg_b2 = _make_segsum()(y2a, y2b, src2, dst1, zrow)
    s2 = _mid_s(s1, agg_a, agg_b, c0, c1, W2_self, b2)
    return _fin(s2, agg_a2, agg_b2, c0, c1)


# speedup vs baseline: 1.2439x; 1.2439x over previous
import functools

import jax
import jax.numpy as jnp
from jax import lax
from jax.experimental import pallas as pl
from jax.experimental.pallas import tpu as pltpu
from jax.experimental.pallas import tpu_sc as plsc

N = 10000
E = 160000
D = 256
DH = 128
NSC = 16
N_PAD = 10240
R = N_PAD // NSC
EPT = E // NSC
B = 80
NCHUNK = EPT // B

BM = 400


def _mm_y_body(x_ref, wn_ref, ya_ref, yb_ref):
    y = jnp.dot(x_ref[...], wn_ref[...], preferred_element_type=jnp.float32)
    ya_ref[...] = y[:, :DH]
    yb_ref[...] = y[:, DH:]


def _mm_y(x, wn):
    return pl.pallas_call(
        _mm_y_body,
        grid=(N // BM,),
        in_specs=[
            pl.BlockSpec((BM, D), lambda i: (i, 0)),
            pl.BlockSpec((D, D), lambda i: (0, 0)),
        ],
        out_specs=[
            pl.BlockSpec((BM, DH), lambda i: (i, 0)),
            pl.BlockSpec((BM, DH), lambda i: (i, 0)),
        ],
        out_shape=[
            jax.ShapeDtypeStruct((N_PAD, DH), jnp.float32),
            jax.ShapeDtypeStruct((N_PAD, DH), jnp.float32),
        ],
    )(x, wn)


def _mm_s_body(x_ref, ws_ref, b_ref, s_ref):
    s_ref[...] = jnp.dot(x_ref[...], ws_ref[...], preferred_element_type=jnp.float32) + b_ref[...]


def _mm_s(x, ws, b):
    return pl.pallas_call(
        _mm_s_body,
        grid=(N // BM,),
        in_specs=[
            pl.BlockSpec((BM, D), lambda i: (i, 0)),
            pl.BlockSpec((D, D), lambda i: (0, 0)),
            pl.BlockSpec((1, D), lambda i: (0, 0)),
        ],
        out_specs=pl.BlockSpec((BM, D), lambda i: (i, 0)),
        out_shape=jax.ShapeDtypeStruct((N_PAD, D), jnp.float32),
    )(x, ws, b.reshape(1, D))


def _h_of(s1_ref, aa_ref, ab_ref, c0_ref, c1_ref):
    inv = 1.0 / jnp.maximum(c0_ref[...] + c1_ref[...], 1.0)
    agg = jnp.concatenate([aa_ref[...], ab_ref[...]], axis=1) * inv
    return jnp.maximum(s1_ref[...] + agg, 0.0)


def _mid_y_body(s1_ref, aa_ref, ab_ref, c0_ref, c1_ref, wn_ref, ya_ref, yb_ref):
    h = _h_of(s1_ref, aa_ref, ab_ref, c0_ref, c1_ref)
    y = jnp.dot(h, wn_ref[...], preferred_element_type=jnp.float32)
    ya_ref[...] = y[:, :DH]
    yb_ref[...] = y[:, DH:]


def _mid_s_body(s1_ref, aa_ref, ab_ref, c0_ref, c1_ref, ws_ref, b_ref, s2_ref):
    h = _h_of(s1_ref, aa_ref, ab_ref, c0_ref, c1_ref)
    s2_ref[...] = jnp.dot(h, ws_ref[...], preferred_element_type=jnp.float32) + b_ref[...]


_MID_IN = [
    pl.BlockSpec((BM, D), lambda i: (i, 0)),
    pl.BlockSpec((BM, DH), lambda i: (i, 0)),
    pl.BlockSpec((BM, DH), lambda i: (i, 0)),
    pl.BlockSpec((BM, 1), lambda i: (i, 0)),
    pl.BlockSpec((BM, 1), lambda i: (i, 0)),
    pl.BlockSpec((D, D), lambda i: (0, 0)),
]


def _mid_y(s1, aa, ab, c0, c1, wn):
    return pl.pallas_call(
        _mid_y_body,
        grid=(N // BM,),
        in_specs=_MID_IN,
        out_specs=[
            pl.BlockSpec((BM, DH), lambda i: (i, 0)),
            pl.BlockSpec((BM, DH), lambda i: (i, 0)),
        ],
        out_shape=[
            jax.ShapeDtypeStruct((N_PAD, DH), jnp.float32),
            jax.ShapeDtypeStruct((N_PAD, DH), jnp.float32),
        ],
    )(s1, aa, ab, c0, c1, wn)


def _mid_s(s1, aa, ab, c0, c1, ws, b):
    return pl.pallas_call(
        _mid_s_body,
        grid=(N // BM,),
        in_specs=_MID_IN + [pl.BlockSpec((1, D), lambda i: (0, 0))],
        out_specs=pl.BlockSpec((BM, D), lambda i: (i, 0)),
        out_shape=jax.ShapeDtypeStruct((N_PAD, D), jnp.float32),
    )(s1, aa, ab, c0, c1, ws, b.reshape(1, D))


def _fin_body(s2_ref, aa_ref, ab_ref, c0_ref, c1_ref, o_ref):
    inv = 1.0 / jnp.maximum(c0_ref[...] + c1_ref[...], 1.0)
    o_ref[...] = s2_ref[...] + jnp.concatenate([aa_ref[...], ab_ref[...]], axis=1) * inv


def _fin(s2, aa, ab, c0, c1):
    return pl.pallas_call(
        _fin_body,
        grid=(N // BM,),
        in_specs=[
            pl.BlockSpec((BM, D), lambda i: (i, 0)),
            pl.BlockSpec((BM, DH), lambda i: (i, 0)),
            pl.BlockSpec((BM, DH), lambda i: (i, 0)),
            pl.BlockSpec((BM, 1), lambda i: (i, 0)),
            pl.BlockSpec((BM, 1), lambda i: (i, 0)),
        ],
        out_specs=pl.BlockSpec((BM, D), lambda i: (i, 0)),
        out_shape=jax.ShapeDtypeStruct((N, D), jnp.float32),
    )(s2, aa, ab, c0, c1)


ECNT = E // 32


@functools.lru_cache(maxsize=None)
def _make_segsum():
    mesh = plsc.VectorSubcoreMesh(
        core_axis_name="c", subcore_axis_name="s", num_cores=2, num_subcores=NSC)

    DP = 4
    IP = 6

    out_type = [
        jax.ShapeDtypeStruct((N_PAD, DH), jnp.float32),
        jax.ShapeDtypeStruct((N_PAD, DH), jnp.float32),
    ]
    scratch = [
        pltpu.VMEM((IP, B), jnp.int32),
        pltpu.VMEM((IP, B), jnp.int32),
        pltpu.VMEM((DP, B, DH), jnp.float32),
        pltpu.VMEM_SHARED((N_PAD, DH), jnp.float32),
        pltpu.SemaphoreType.DMA,
        pltpu.SemaphoreType.DMA,
        pltpu.SemaphoreType.DMA,
        pltpu.SemaphoreType.DMA,
    ]

    def body(ya, yb, ei, agg_a, agg_b,
             sidx, didx, rows, accum, sisem, disem, gsem, ssem):
        c = lax.axis_index("c")
        s = lax.axis_index("s")
        row0 = s * R
        ebase = s * EPT

        def _zr(r, carry):
            for k in range(DH // 16):
                rows[0, r, pl.ds(k * 16, 16)] = jnp.zeros((16,), jnp.float32)
            return carry
        lax.fori_loop(0, B, _zr, 0)
        for q in range(R // B):
            pltpu.sync_copy(rows.at[0], accum.at[pl.ds(row0 + q * B, B)])
        plsc.subcore_barrier()

        def load_idx(i):
            buf = lax.rem(i, IP)
            off = ebase + i * B
            pltpu.async_copy(ei.at[pl.ds(off, B)], sidx.at[buf], sisem)
            pltpu.async_copy(ei.at[pl.ds(E + off, B)], didx.at[buf], disem)

        def gather(i):
            buf = lax.rem(i, DP)

            @pl.when(c == 0)
            def _():
                pltpu.async_copy(ya.at[sidx.at[lax.rem(i, IP)]], rows.at[buf], gsem)

            @pl.when(c == 1)
            def _():
                pltpu.async_copy(yb.at[sidx.at[lax.rem(i, IP)]], rows.at[buf], gsem)

        def wait_idx(sem):
            pltpu.make_async_copy(ei.at[pl.ds(0, B)], sidx.at[0], sem).wait()

        def wait_gather():
            pltpu.make_async_copy(ya.at[sidx.at[0]], rows.at[0], gsem).wait()

        def wait_scatter():
            pltpu.make_async_copy(rows.at[0], accum.at[didx.at[0]], ssem).wait()

        for k in range(4):
            load_idx(k)
        wait_idx(sisem)
        gather(0)
        wait_idx(sisem)
        gather(1)

        def step(i, carry):
            @pl.when(i >= 2)
            def _():
                wait_scatter()

            @pl.when(i + 4 < NCHUNK)
            def _():
                load_idx(i + 4)

            @pl.when(i + 2 < NCHUNK)
            def _():
                wait_idx(sisem)
                gather(i + 2)

            wait_gather()
            wait_idx(disem)
            pltpu.async_copy(rows.at[lax.rem(i, DP)],
                             accum.at[didx.at[lax.rem(i, IP)]], ssem, add=True)
            return carry

        lax.fori_loop(0, NCHUNK, step, 0)
        wait_scatter()
        wait_scatter()
        plsc.subcore_barrier()

        @pl.when(c == 0)
        def _():
            pltpu.sync_copy(accum.at[pl.ds(row0, R)], agg_a.at[pl.ds(row0, R)])

        @pl.when(c == 1)
        def _():
            pltpu.sync_copy(accum.at[pl.ds(row0, R)], agg_b.at[pl.ds(row0, R)])

    return pl.kernel(
        body, out_type=out_type, mesh=mesh, scratch_types=scratch,
        compiler_params=pltpu.CompilerParams(needs_layout_passes=False))


@functools.lru_cache(maxsize=None)
def _make_cnt():
    mesh = plsc.VectorSubcoreMesh(
        core_axis_name="c", subcore_axis_name="s", num_cores=2, num_subcores=NSC)

    out_type = [
        jax.ShapeDtypeStruct((N_PAD,), jnp.float32),
        jax.ShapeDtypeStruct((N_PAD,), jnp.float32),
    ]
    scratch = [
        pltpu.VMEM((ECNT,), jnp.int32),
        pltpu.VMEM((N_PAD,), jnp.float32),
        pltpu.VMEM((NSC, R), jnp.float32),
        pltpu.VMEM((R,), jnp.float32),
        pltpu.VMEM_SHARED((NSC, N_PAD), jnp.float32),
    ]

    def body(ei, c0_out, c1_out, didx, cntp, cred, cout, cnt_all):
        c = lax.axis_index("c")
        s = lax.axis_index("s")
        w = c * NSC + s
        row0 = s * R

        pltpu.sync_copy(ei.at[pl.ds(E + w * ECNT, ECNT)], didx)

        def _zc(i, carry):
            cntp[pl.ds(i * 16, 16)] = jnp.zeros((16,), jnp.float32)
            return carry
        lax.fori_loop(0, N_PAD // 16, _zc, 0)

        ones16 = jnp.ones((16,), jnp.float32)

        def _cc(i, carry):
            d16 = didx[pl.ds(i * 16, 16)]
            plsc.addupdate_scatter(cntp, [d16], ones16)
            return carry
        lax.fori_loop(0, ECNT // 16, _cc, 0)
        d16 = didx[pl.ds(ECNT - 16, 16)]
        lanes = lax.broadcasted_iota(jnp.int32, (16,), 0)
        plsc.addupdate_scatter(cntp, [d16], ones16, mask=lanes >= 8)

        pltpu.sync_copy(cntp, cnt_all.at[s])
        plsc.subcore_barrier()
        pltpu.sync_copy(cnt_all.at[:, pl.ds(row0, R)], cred)

        def red(j, carry):
            acc = jnp.zeros((16,), jnp.float32)
            for r in range(NSC):
                acc = acc + cred[r, pl.ds(j * 16, 16)]
            cout[pl.ds(j * 16, 16)] = acc
            return carry
        lax.fori_loop(0, R // 16, red, 0)

        @pl.when(c == 0)
        def _():
            pltpu.sync_copy(cout, c0_out.at[pl.ds(row0, R)])

        @pl.when(c == 1)
        def _():
            pltpu.sync_copy(cout, c1_out.at[pl.ds(row0, R)])

    return pl.kernel(
        body, out_type=out_type, mesh=mesh, scratch_types=scratch,
        compiler_params=pltpu.CompilerParams(needs_layout_passes=False))


@jax.jit
def kernel(x, edge_index, W1_self, W1_neigh, b1, W2_self, W2_neigh, b2):
    ei = edge_index.reshape(2 * E)
    cnt0, cnt1 = _make_cnt()(ei)
    c0 = cnt0.reshape(N_PAD, 1)
    c1 = cnt1.reshape(N_PAD, 1)
    y1a, y1b = _mm_y(x, W1_neigh)
    agg_a, agg_b = _make_segsum()(y1a, y1b, ei)
    s1 = _mm_s(x, W1_self, b1)
    y2a, y2b = _mid_y(s1, agg_a, agg_b, c0, c1, W2_neigh)
    agg_a2, agg_b2 = _make_segsum()(y2a, y2b, ei)
    s2 = _mid_s(s1, agg_a, agg_b, c0, c1, W2_self, b2)
    return _fin(s2, agg_a2, agg_b2, c0, c1)
